# baseline (device time: 476227 ns/iter reference)
import jax
import jax.numpy as jnp
from jax import lax
from jax.experimental import pallas as pl
from jax.experimental.pallas import tpu as pltpu

M = 4096
N = 8192
K = 4096
HALF = 2048
HB = 512

C = 8
CH = HB // C
R = 256
CPS = R // CH
T = 4
BK = 256
NK = K // BK

_MESH = pl.DeviceIdType.MESH

_COMM = True


def _fused_body(
    s_ref,
    x_ref,
    dy_ref,
    o_ref,
    yb_ref,
    acc_c,
    acc_o,
    vb_ref,
    l1,
    l2,
    ya_s,
    ya_r,
    xp_s,
    xp_r,
    zp_s,
    zp_r,
    xr_s,
    xr_r,
    zr_s,
    zr_r,
):
    del s_ref
    t = pl.program_id(0)
    k = pl.program_id(1)
    r = t // 2
    is_cross = (t % 2) == 0

    ix = lax.axis_index("x")
    iy = lax.axis_index("y")
    iz = lax.axis_index("z")
    y_nbr = (ix, 1 - iy, iz)
    x_nbr = (1 - ix, iy, iz)
    z_nbr = (ix, iy, 1 - iz)

    q = 2 * ix + iz
    off = q * HB
    off_x = (2 * (1 - ix) + iz) * HB
    off_z = (2 * ix + (1 - iz)) * HB
    off_d = (2 * (1 - ix) + (1 - iz)) * HB

    def y_edge(c):
        return pltpu.make_async_remote_copy(
            src_ref=acc_c.at[pl.ds((c % CPS) * CH, CH)],
            dst_ref=yb_ref.at[pl.ds(c * CH, CH)],
            send_sem=ya_s.at[c],
            recv_sem=ya_r.at[c],
            device_id=y_nbr,
            device_id_type=_MESH,
        )

    def prim_edge(c, send, recv, dev):
        return pltpu.make_async_remote_copy(
            src_ref=acc_o.at[c // CPS, pl.ds((c % CPS) * CH, CH)],
            dst_ref=o_ref.at[pl.ds(off + c * CH, CH)],
            send_sem=send.at[c],
            recv_sem=recv.at[c],
            device_id=dev,
            device_id_type=_MESH,
        )

    def o_edge(row_off, send, recv, c, dev):
        sl = pl.ds(row_off + c * CH, CH)
        return pltpu.make_async_remote_copy(
            src_ref=o_ref.at[sl],
            dst_ref=o_ref.at[sl],
            send_sem=send.at[c],
            recv_sem=recv.at[c],
            device_id=dev,
            device_id_type=_MESH,
        )

    def local_store(c):
        return pltpu.make_async_copy(
            acc_o.at[c // CPS, pl.ds((c % CPS) * CH, CH)],
            o_ref.at[pl.ds(off + c * CH, CH)],
            l2.at[c],
        )

    if _COMM:

        @pl.when((t == 0) & (k == 0))
        def _():
            barrier = pltpu.get_barrier_semaphore()
            for nbr in (x_nbr, y_nbr, z_nbr):
                pl.semaphore_signal(
                    barrier, inc=1, device_id=nbr, device_id_type=_MESH
                )
            pl.semaphore_wait(barrier, 3)

    prod = lax.dot_general(
        x_ref[...],
        dy_ref[...],
        dimension_numbers=(((0,), (0,)), ((), ())),
        preferred_element_type=jnp.float32,
    )

    @pl.when(is_cross)
    def _():
        @pl.when(k == 0)
        def _():
            if _COMM:

                @pl.when(r >= 1)
                def _():
                    for cc in range(CPS):
                        y_edge(cc).wait_send()

            acc_c[...] = prod

        @pl.when(k != 0)
        def _():
            acc_c[...] += prod

    @pl.when(jnp.logical_not(is_cross))
    def _():
        @pl.when(k == 0)
        def _():
            acc_o[r, :, :] = prod

        @pl.when(k != 0)
        def _():
            acc_o[r, :, :] += prod

    @pl.when((k == NK - 1) & is_cross)
    def _():
        if _COMM:
            for cc in range(CPS):
                y_edge(CPS * r + cc).start()

    @pl.when((k == NK - 1) & jnp.logical_not(is_cross))
    def _():
        for cc in range(CPS):
            c = CPS * r + cc
            if _COMM:
                y_edge(c).wait_recv()
            cp = pltpu.make_async_copy(
                yb_ref.at[pl.ds(c * CH, CH)], vb_ref, l1
            )
            cp.start()
            cp.wait()
            acc_o[r, pl.ds(cc * CH, CH), :] += vb_ref[...]
            local_store(c).start()
            if _COMM:
                prim_edge(c, xp_s, xp_r, x_nbr).start()
                prim_edge(c, zp_s, zp_r, z_nbr).start()

    @pl.when((t == T - 1) & (k == NK - 1))
    def _():
        if _COMM:
            relays = []
            for c in range(C // 2):
                o_edge(off_x, xp_s, xp_r, c, x_nbr).wait_recv()
                rr = o_edge(off_x, zr_s, zr_r, c, z_nbr)
                rr.start()
                relays.append(rr)
            for c in range(C // 2, C):
                o_edge(off_z, zp_s, zp_r, c, z_nbr).wait_recv()
                rr = o_edge(off_z, xr_s, xr_r, c, x_nbr)
                rr.start()
                relays.append(rr)
            for c in range(C // 2, C):
                o_edge(off_x, xp_s, xp_r, c, x_nbr).wait_recv()
            for c in range(C // 2):
                o_edge(off_z, zp_s, zp_r, c, z_nbr).wait_recv()
            for c in range(C // 2):
                o_edge(off_d, zr_s, zr_r, c, z_nbr).wait_recv()
            for c in range(C // 2, C):
                o_edge(off_d, xr_s, xr_r, c, x_nbr).wait_recv()
            for c in range(CPS, C):
                y_edge(c).wait_send()
            for c in range(C):
                prim_edge(c, xp_s, xp_r, x_nbr).wait_send()
                prim_edge(c, zp_s, zp_r, z_nbr).wait_send()
            for rr in relays:
                rr.wait_send()
        for c in range(C):
            local_store(c).wait()


def _fused(s, x, dy):
    return pl.pallas_call(
        _fused_body,
        grid_spec=pltpu.PrefetchScalarGridSpec(
            num_scalar_prefetch=1,
            grid=(T, NK),
            in_specs=[
                pl.BlockSpec((BK, R), lambda t, k, s: (k, s[t])),
                pl.BlockSpec((BK, N), lambda t, k, s: (k, 0)),
            ],
            out_specs=[
                pl.BlockSpec(memory_space=pl.ANY),
                pl.BlockSpec(memory_space=pl.ANY),
            ],
            scratch_shapes=[
                pltpu.VMEM((R, N), jnp.float32),
                pltpu.VMEM((2, R, N), jnp.float32),
                pltpu.VMEM((CH, N), jnp.float32),
                pltpu.SemaphoreType.DMA,
                pltpu.SemaphoreType.DMA((C,)),
                pltpu.SemaphoreType.DMA((C,)),
                pltpu.SemaphoreType.DMA((C,)),
                pltpu.SemaphoreType.DMA((C,)),
                pltpu.SemaphoreType.DMA((C,)),
                pltpu.SemaphoreType.DMA((C,)),
                pltpu.SemaphoreType.DMA((C,)),
                pltpu.SemaphoreType.DMA((C,)),
                pltpu.SemaphoreType.DMA((C,)),
                pltpu.SemaphoreType.DMA((C,)),
                pltpu.SemaphoreType.DMA((C,)),
            ],
        ),
        out_shape=[
            jax.ShapeDtypeStruct((HALF, N), jnp.float32),
            jax.ShapeDtypeStruct((HB, N), jnp.float32),
        ],
        compiler_params=pltpu.CompilerParams(
            dimension_semantics=("arbitrary", "arbitrary"),
            collective_id=0 if _COMM else None,
            vmem_limit_bytes=56 * 1024 * 1024,
        ),
    )(s, x, dy)


def kernel(x, dy):
    ix = lax.axis_index("x")
    iy = lax.axis_index("y")
    iz = lax.axis_index("z")
    q = 2 * ix + iz
    h1 = 4 * iy + q
    h2 = 4 * (1 - iy) + q
    s = jnp.stack([2 * h2, 2 * h1, 2 * h2 + 1, 2 * h1 + 1]).astype(jnp.int32)
    out, _ = _fused(s, x, dy)
    return out
